# baseline (speedup 1.0000x reference)
import functools

import jax
import jax.numpy as jnp
from jax import lax
from jax.experimental import pallas as pl
from jax.experimental.pallas import tpu as pltpu
from jax.experimental.pallas import tpu_sc as plsc

_TN = 200
_CHUNK = 128
_NBUF = 8
_LEAD = 4
_NW = 32


def _sc_gather_body(n_frames, n_beads, edges_per_frame, ch_core0, ch_core1,
                    nl_ref, feat_ref, out_ref, idx_m, rows_v, *sems):
    gsem = sems[:_NBUF]
    ssem = sems[_NBUF:]

    cax = lax.axis_index("c")
    sax = lax.axis_index("s")

    def _do(base_chunk, count):
        pltpu.sync_copy(nl_ref.at[pl.ds(base_chunk, count)],
                        idx_m.at[pl.ds(0, count)])

        def _fix_chunk(j, _):
            frame = jnp.minimum((base_chunk + j) * _CHUNK
                                // edges_per_frame, n_frames - 1)
            off = frame * n_beads
            def _fix_slice(l, _):
                sl = pl.ds(l * 16, 16)
                idx_m[j, sl] = idx_m[j, sl] + off
                return 0
            lax.fori_loop(0, _CHUNK // 16, _fix_slice, 0)
            return 0

        lax.fori_loop(0, count, _fix_chunk, 0)

        def _gather_start(c, b):
            pltpu.async_copy(feat_ref.at[idx_m.at[c]], rows_v.at[b],
                             gsem[b])

        def _gather_wait(b):
            pltpu.make_async_copy(feat_ref.at[pl.ds(0, _CHUNK)],
                                  rows_v.at[b], gsem[b]).wait()

        def _scatter_start(c, b):
            pltpu.async_copy(
                rows_v.at[b],
                out_ref.at[pl.ds((base_chunk + c) * _CHUNK, _CHUNK)],
                ssem[b])

        def _scatter_wait(b):
            pltpu.make_async_copy(rows_v.at[b],
                                  out_ref.at[pl.ds(0, _CHUNK)],
                                  ssem[b]).wait()

        for c in range(_LEAD):
            _gather_start(c, c)

        def _round(r, _):
            for b in range(_NBUF):
                c = r * _NBUF + b
                b2 = (b + _LEAD) % _NBUF

                @pl.when(c + _LEAD < count)
                def _():
                    @pl.when(c >= _NBUF - _LEAD)
                    def _():
                        _scatter_wait(b2)
                    _gather_start(c + _LEAD, b2)

                _gather_wait(b)
                _scatter_start(c, b)
            return 0

        lax.fori_loop(0, count // _NBUF, _round, 0)
        for c in range(count - _LEAD, count):
            _scatter_wait(c % _NBUF)

    @pl.when(cax == 0)
    def _():
        _do(sax * ch_core0, ch_core0)

    @pl.when(cax == 1)
    def _():
        _do(16 * ch_core0 + sax * ch_core1, ch_core1)


def _tc_body(nl_unused, rbf_ref, mask_ref, gath_ref, w1_ref, b1_ref,
             w2_ref, b2_ref, out_ref):
    tn, k, g_dim = rbf_ref.shape[1], rbf_ref.shape[2], rbf_ref.shape[3]
    f = out_ref.shape[2]
    rows = tn * k

    rbf = rbf_ref[0].reshape(rows, g_dim).astype(jnp.bfloat16)
    h = jnp.dot(rbf, w1_ref[...], preferred_element_type=jnp.float32)
    h = h + b1_ref[...]
    h = jax.nn.softplus(h) - jnp.log(2.0)
    filt = jnp.dot(h.astype(jnp.bfloat16), w2_ref[...],
                   preferred_element_type=jnp.float32)
    filt = filt + b2_ref[...]

    fw = f // 2
    packed = gath_ref[...]
    g_lo = lax.bitcast_convert_type(packed << 16, jnp.float32)
    g_hi = lax.bitcast_convert_type(
        packed & jnp.int32(-65536), jnp.float32)
    mask3 = lax.broadcast_in_dim(mask_ref[0], (tn, k, fw), (0, 1))
    out_lo = ((filt[:, :fw] * g_lo).reshape(tn, k, fw) * mask3).sum(axis=1)
    out_hi = ((filt[:, fw:] * g_hi).reshape(tn, k, fw) * mask3).sum(axis=1)
    out_ref[0] = jnp.concatenate([out_lo, out_hi], axis=1)


def kernel(features, rbf_expansion, neighbor_list, neighbor_mask,
           W1, b1, W2, b2):
    B, N, F = features.shape
    _, _, K, G = rbf_expansion.shape
    tn = _TN
    edges = B * N * K
    total_chunks = -(-edges // (_NW * _NBUF * _CHUNK)) * (_NW * _NBUF)
    ch_core0 = total_chunks // 5 // 16 // _NBUF * _NBUF
    ch_core1 = (total_chunks - 16 * ch_core0) // 16
    assert ch_core1 % _NBUF == 0 and 16 * (ch_core0 + ch_core1) == total_chunks
    edges_pad = total_chunks * _CHUNK
    fw = F // 2

    nl_flat = neighbor_list.reshape(-1)
    nl_pad = jnp.concatenate(
        [nl_flat, jnp.zeros((edges_pad - edges,), jnp.int32)])
    nl2d = nl_pad.reshape(edges_pad // _CHUNK, _CHUNK)
    feat_bf = features.astype(jnp.bfloat16).reshape(B * N, F)
    table = lax.bitcast_convert_type(
        jnp.stack([feat_bf[:, :fw], feat_bf[:, fw:]], axis=-1), jnp.int32)

    mesh = plsc.VectorSubcoreMesh(core_axis_name="c", subcore_axis_name="s")
    gathered = pl.kernel(
        functools.partial(_sc_gather_body, B, N, N * K, ch_core0, ch_core1),
        out_type=jax.ShapeDtypeStruct((edges_pad, fw), jnp.int32),
        mesh=mesh,
        compiler_params=pltpu.CompilerParams(use_tc_tiling_on_sc=False),
        scratch_types=(
            [pltpu.VMEM((max(ch_core0, ch_core1), _CHUNK), jnp.int32),
             pltpu.VMEM((_NBUF, _CHUNK, fw), jnp.int32)]
            + [pltpu.SemaphoreType.DMA] * (2 * _NBUF)),
    )(nl2d, table)

    b1r = b1.reshape(1, F)
    b2r = b2.reshape(1, F)
    w1 = W1.astype(jnp.bfloat16)
    w2 = W2.astype(jnp.bfloat16)
    rows = tn * K

    return pl.pallas_call(
        _tc_body,
        grid=(B, N // tn),
        in_specs=[
            pl.BlockSpec((1, tn, K), lambda b, t: (b, t, 0)),
            pl.BlockSpec((1, tn, K, G), lambda b, t: (b, t, 0, 0)),
            pl.BlockSpec((1, tn, K), lambda b, t: (b, t, 0)),
            pl.BlockSpec((rows, fw),
                         lambda b, t, nt=N // tn: (b * nt + t, 0)),
            pl.BlockSpec((G, F), lambda b, t: (0, 0)),
            pl.BlockSpec((1, F), lambda b, t: (0, 0)),
            pl.BlockSpec((F, F), lambda b, t: (0, 0)),
            pl.BlockSpec((1, F), lambda b, t: (0, 0)),
        ],
        out_specs=pl.BlockSpec((1, tn, F), lambda b, t: (b, t, 0)),
        out_shape=jax.ShapeDtypeStruct((B, N, F), jnp.float32),
    )(neighbor_list, rbf_expansion, neighbor_mask, gathered,
      w1, b1r, w2, b2r)


# speedup vs baseline: 1.0099x; 1.0099x over previous
import functools

import jax
import jax.numpy as jnp
from jax import lax
from jax.experimental import pallas as pl
from jax.experimental.pallas import tpu as pltpu
from jax.experimental.pallas import tpu_sc as plsc

_TN = 200
_CHUNK = 128
_NBUF = 8
_LEAD = 4
_NW = 32


def _sc_gather_body(n_frames, n_beads, edges_per_frame, ch_core0, ch_core1,
                    nl_ref, feat_ref, out_ref, idx_m, rows_v, *sems):
    gsem = sems[:_NBUF]
    ssem = sems[_NBUF:]

    cax = lax.axis_index("c")
    sax = lax.axis_index("s")

    def _do(base_chunk, count):
        pltpu.sync_copy(nl_ref.at[pl.ds(base_chunk, count)],
                        idx_m.at[pl.ds(0, count)])

        def _fix_chunk(j, _):
            frame = jnp.minimum((base_chunk + j) * _CHUNK
                                // edges_per_frame, n_frames - 1)
            off = frame * n_beads
            def _fix_slice(l, _):
                sl = pl.ds(l * 16, 16)
                idx_m[j, sl] = idx_m[j, sl] + off
                return 0
            lax.fori_loop(0, _CHUNK // 16, _fix_slice, 0)
            return 0

        lax.fori_loop(0, count, _fix_chunk, 0)

        def _gather_start(c, b):
            pltpu.async_copy(feat_ref.at[idx_m.at[c]], rows_v.at[b],
                             gsem[b])

        def _gather_wait(b):
            pltpu.make_async_copy(feat_ref.at[pl.ds(0, _CHUNK)],
                                  rows_v.at[b], gsem[b]).wait()

        def _scatter_start(c, b):
            pltpu.async_copy(
                rows_v.at[b],
                out_ref.at[pl.ds((base_chunk + c) * _CHUNK, _CHUNK)],
                ssem[b])

        def _scatter_wait(b):
            pltpu.make_async_copy(rows_v.at[b],
                                  out_ref.at[pl.ds(0, _CHUNK)],
                                  ssem[b]).wait()

        for c in range(_LEAD):
            _gather_start(c, c)

        def _round(r, _):
            for b in range(_NBUF):
                c = r * _NBUF + b
                b2 = (b + _LEAD) % _NBUF

                @pl.when(c + _LEAD < count)
                def _():
                    @pl.when(c >= _NBUF - _LEAD)
                    def _():
                        _scatter_wait(b2)
                    _gather_start(c + _LEAD, b2)

                _gather_wait(b)
                _scatter_start(c, b)
            return 0

        lax.fori_loop(0, count // _NBUF, _round, 0)
        for c in range(count - _LEAD, count):
            _scatter_wait(c % _NBUF)

    @pl.when(cax == 0)
    def _():
        _do(sax * ch_core0, ch_core0)

    @pl.when(cax == 1)
    def _():
        _do(16 * ch_core0 + sax * ch_core1, ch_core1)


def _tc_body(nl_unused, rbf_ref, mask_ref, gath_ref, w1_ref, b1_ref,
             w2_ref, b2_ref, out_ref):
    tn, k, g_dim = rbf_ref.shape[1], rbf_ref.shape[2], rbf_ref.shape[3]
    f = out_ref.shape[2]
    rows = tn * k

    rbf = rbf_ref[0].reshape(rows, g_dim)
    h = jnp.dot(rbf, w1_ref[...], preferred_element_type=jnp.float32)
    h = h + b1_ref[...]
    h = jax.nn.softplus(h) - jnp.log(2.0)
    filt = jnp.dot(h.astype(jnp.bfloat16), w2_ref[...],
                   preferred_element_type=jnp.float32)
    filt = filt + b2_ref[...]

    fw = f // 2
    packed = gath_ref[...]
    g_lo = lax.bitcast_convert_type(packed << 16, jnp.float32)
    g_hi = lax.bitcast_convert_type(
        packed & jnp.int32(-65536), jnp.float32)
    mask3 = lax.broadcast_in_dim(mask_ref[0], (tn, k, fw), (0, 1))
    out_lo = ((filt[:, :fw] * g_lo).reshape(tn, k, fw) * mask3).sum(axis=1)
    out_hi = ((filt[:, fw:] * g_hi).reshape(tn, k, fw) * mask3).sum(axis=1)
    out_ref[0] = jnp.concatenate([out_lo, out_hi], axis=1)


def kernel(features, rbf_expansion, neighbor_list, neighbor_mask,
           W1, b1, W2, b2):
    B, N, F = features.shape
    _, _, K, G = rbf_expansion.shape
    tn = _TN
    edges = B * N * K
    total_chunks = -(-edges // (_NW * _NBUF * _CHUNK)) * (_NW * _NBUF)
    ch_core0 = total_chunks // 5 // 16 // _NBUF * _NBUF
    ch_core1 = (total_chunks - 16 * ch_core0) // 16
    assert ch_core1 % _NBUF == 0 and 16 * (ch_core0 + ch_core1) == total_chunks
    edges_pad = total_chunks * _CHUNK
    fw = F // 2

    nl_flat = neighbor_list.reshape(-1)
    nl_pad = jnp.concatenate(
        [nl_flat, jnp.zeros((edges_pad - edges,), jnp.int32)])
    nl2d = nl_pad.reshape(edges_pad // _CHUNK, _CHUNK)
    feat_bf = features.astype(jnp.bfloat16).reshape(B * N, F)
    table = lax.bitcast_convert_type(
        jnp.stack([feat_bf[:, :fw], feat_bf[:, fw:]], axis=-1), jnp.int32)

    mesh = plsc.VectorSubcoreMesh(core_axis_name="c", subcore_axis_name="s")
    gathered = pl.kernel(
        functools.partial(_sc_gather_body, B, N, N * K, ch_core0, ch_core1),
        out_type=jax.ShapeDtypeStruct((edges_pad, fw), jnp.int32),
        mesh=mesh,
        compiler_params=pltpu.CompilerParams(use_tc_tiling_on_sc=False),
        scratch_types=(
            [pltpu.VMEM((max(ch_core0, ch_core1), _CHUNK), jnp.int32),
             pltpu.VMEM((_NBUF, _CHUNK, fw), jnp.int32)]
            + [pltpu.SemaphoreType.DMA] * (2 * _NBUF)),
    )(nl2d, table)

    b1r = b1.reshape(1, F)
    b2r = b2.reshape(1, F)
    w1 = W1.astype(jnp.bfloat16)
    w2 = W2.astype(jnp.bfloat16)
    rows = tn * K

    return pl.pallas_call(
        _tc_body,
        grid=(B, N // tn),
        in_specs=[
            pl.BlockSpec((1, tn, K), lambda b, t: (b, t, 0)),
            pl.BlockSpec((1, tn, K, G), lambda b, t: (b, t, 0, 0)),
            pl.BlockSpec((1, tn, K), lambda b, t: (b, t, 0)),
            pl.BlockSpec((rows, fw),
                         lambda b, t, nt=N // tn: (b * nt + t, 0)),
            pl.BlockSpec((G, F), lambda b, t: (0, 0)),
            pl.BlockSpec((1, F), lambda b, t: (0, 0)),
            pl.BlockSpec((F, F), lambda b, t: (0, 0)),
            pl.BlockSpec((1, F), lambda b, t: (0, 0)),
        ],
        out_specs=pl.BlockSpec((1, tn, F), lambda b, t: (b, t, 0)),
        out_shape=jax.ShapeDtypeStruct((B, N, F), jnp.float32),
    )(neighbor_list, rbf_expansion.astype(jnp.bfloat16), neighbor_mask,
      gathered, w1, b1r, w2, b2r)


# speedup vs baseline: 1.0141x; 1.0042x over previous
import functools

import jax
import jax.numpy as jnp
from jax import lax
from jax.experimental import pallas as pl
from jax.experimental.pallas import tpu as pltpu
from jax.experimental.pallas import tpu_sc as plsc

_TN = 200
_CHUNK = 128
_NBUF = 8
_LEAD = 4
_NW = 32


def _sc_gather_body(n_frames, n_beads, edges_per_frame, ch_core0, ch_core1,
                    nl_ref, feat_ref, out_ref, idx_m, rows_v, *sems):
    gsem = sems[:_NBUF]
    ssem = sems[_NBUF:]

    cax = lax.axis_index("c")
    sax = lax.axis_index("s")

    def _do(base_chunk, count):
        pltpu.sync_copy(nl_ref.at[pl.ds(base_chunk, count)],
                        idx_m.at[pl.ds(0, count)])

        def _fix_chunk(j, _):
            frame = jnp.minimum((base_chunk + j) * _CHUNK
                                // edges_per_frame, n_frames - 1)
            off = frame * n_beads
            def _fix_slice(l, _):
                sl = pl.ds(l * 16, 16)
                idx_m[j, sl] = idx_m[j, sl] + off
                return 0
            lax.fori_loop(0, _CHUNK // 16, _fix_slice, 0)
            return 0

        lax.fori_loop(0, count, _fix_chunk, 0)

        def _gather_start(c, b):
            pltpu.async_copy(feat_ref.at[idx_m.at[c]], rows_v.at[b],
                             gsem[b])

        def _gather_wait(b):
            pltpu.make_async_copy(feat_ref.at[pl.ds(0, _CHUNK)],
                                  rows_v.at[b], gsem[b]).wait()

        def _scatter_start(c, b):
            pltpu.async_copy(
                rows_v.at[b],
                out_ref.at[pl.ds((base_chunk + c) * _CHUNK, _CHUNK)],
                ssem[b])

        def _scatter_wait(b):
            pltpu.make_async_copy(rows_v.at[b],
                                  out_ref.at[pl.ds(0, _CHUNK)],
                                  ssem[b]).wait()

        for c in range(_LEAD):
            _gather_start(c, c)

        def _round(r, _):
            for b in range(_NBUF):
                c = r * _NBUF + b
                b2 = (b + _LEAD) % _NBUF

                @pl.when(c + _LEAD < count)
                def _():
                    @pl.when(c >= _NBUF - _LEAD)
                    def _():
                        _scatter_wait(b2)
                    _gather_start(c + _LEAD, b2)

                _gather_wait(b)
                _scatter_start(c, b)
            return 0

        lax.fori_loop(0, count // _NBUF, _round, 0)
        for c in range(count - _LEAD, count):
            _scatter_wait(c % _NBUF)

    @pl.when(cax == 0)
    def _():
        _do(sax * ch_core0, ch_core0)

    @pl.when(cax == 1)
    def _():
        _do(16 * ch_core0 + sax * ch_core1, ch_core1)


def _tc_body(nl_unused, rbf_ref, mask_ref, gath_ref, w1_ref, b1_ref,
             w2_ref, b2_ref, out_ref):
    tn, k, g_dim = rbf_ref.shape[1], rbf_ref.shape[2], rbf_ref.shape[3]
    f = out_ref.shape[2]
    rows = tn * k

    rbf = rbf_ref[0].reshape(rows, g_dim)
    h = jnp.dot(rbf, w1_ref[...], preferred_element_type=jnp.float32)
    h = h + b1_ref[...]
    h = jax.nn.softplus(h) - jnp.log(2.0)
    filt = jnp.dot(h.astype(jnp.bfloat16), w2_ref[...],
                   preferred_element_type=jnp.float32)
    filt = filt + b2_ref[...]

    fw = f // 2
    packed = gath_ref[...]
    g_lo = lax.bitcast_convert_type(packed << 16, jnp.float32)
    g_hi = lax.bitcast_convert_type(
        packed & jnp.int32(-65536), jnp.float32)
    mask3 = lax.broadcast_in_dim(mask_ref[0], (tn, k, fw), (0, 1))
    out_lo = ((filt[:, :fw] * g_lo).reshape(tn, k, fw) * mask3).sum(axis=1)
    out_hi = ((filt[:, fw:] * g_hi).reshape(tn, k, fw) * mask3).sum(axis=1)
    out_ref[0] = jnp.concatenate([out_lo, out_hi], axis=1)


def kernel(features, rbf_expansion, neighbor_list, neighbor_mask,
           W1, b1, W2, b2):
    B, N, F = features.shape
    _, _, K, G = rbf_expansion.shape
    tn = _TN
    edges = B * N * K
    total_chunks = -(-edges // (_NW * _NBUF * _CHUNK)) * (_NW * _NBUF)
    ch_core0 = total_chunks // _NW
    ch_core1 = (total_chunks - 16 * ch_core0) // 16
    assert ch_core1 % _NBUF == 0 and 16 * (ch_core0 + ch_core1) == total_chunks
    edges_pad = total_chunks * _CHUNK
    fw = F // 2

    nl_flat = neighbor_list.reshape(-1)
    nl_pad = jnp.concatenate(
        [nl_flat, jnp.zeros((edges_pad - edges,), jnp.int32)])
    nl2d = nl_pad.reshape(edges_pad // _CHUNK, _CHUNK)
    feat_bf = features.astype(jnp.bfloat16).reshape(B * N, F)
    table = lax.bitcast_convert_type(
        jnp.stack([feat_bf[:, :fw], feat_bf[:, fw:]], axis=-1), jnp.int32)

    mesh = plsc.VectorSubcoreMesh(core_axis_name="c", subcore_axis_name="s")
    gathered = pl.kernel(
        functools.partial(_sc_gather_body, B, N, N * K, ch_core0, ch_core1),
        out_type=jax.ShapeDtypeStruct((edges_pad, fw), jnp.int32),
        mesh=mesh,
        compiler_params=pltpu.CompilerParams(use_tc_tiling_on_sc=False),
        scratch_types=(
            [pltpu.VMEM((max(ch_core0, ch_core1), _CHUNK), jnp.int32),
             pltpu.VMEM((_NBUF, _CHUNK, fw), jnp.int32)]
            + [pltpu.SemaphoreType.DMA] * (2 * _NBUF)),
    )(nl2d, table)

    b1r = b1.reshape(1, F)
    b2r = b2.reshape(1, F)
    w1 = W1.astype(jnp.bfloat16)
    w2 = W2.astype(jnp.bfloat16)
    rows = tn * K

    return pl.pallas_call(
        _tc_body,
        grid=(B, N // tn),
        in_specs=[
            pl.BlockSpec((1, tn, K), lambda b, t: (b, t, 0)),
            pl.BlockSpec((1, tn, K, G), lambda b, t: (b, t, 0, 0)),
            pl.BlockSpec((1, tn, K), lambda b, t: (b, t, 0)),
            pl.BlockSpec((rows, fw),
                         lambda b, t, nt=N // tn: (b * nt + t, 0)),
            pl.BlockSpec((G, F), lambda b, t: (0, 0)),
            pl.BlockSpec((1, F), lambda b, t: (0, 0)),
            pl.BlockSpec((F, F), lambda b, t: (0, 0)),
            pl.BlockSpec((1, F), lambda b, t: (0, 0)),
        ],
        out_specs=pl.BlockSpec((1, tn, F), lambda b, t: (b, t, 0)),
        out_shape=jax.ShapeDtypeStruct((B, N, F), jnp.float32),
    )(neighbor_list, rbf_expansion.astype(jnp.bfloat16), neighbor_mask,
      gathered, w1, b1r, w2, b2r)
